# Initial kernel scaffold; baseline (speedup 1.0000x reference)
import jax
import jax.numpy as jnp
from jax.experimental import pallas as pl


def kernel(signal, barycentric, kernels, biases):
    raise NotImplementedError("write your pallas kernel here")


# speedup vs baseline: 6.3597x; 6.3597x over previous
import functools

import jax
import jax.numpy as jnp
from jax import lax
from jax.experimental import pallas as pl
from jax.experimental.pallas import tpu as pltpu
from jax.experimental.pallas import tpu_sc as plsc

_B, _M, _N, _O, _A, _R = 1, 6890, 64, 64, 6, 5
_MPAD = 6912
_T = _R * 3
_NROWS = _A * _M
_NW = 32
_CHUNK = 8
_RPW = ((_NROWS + _NW * _CHUNK - 1) // (_NW * _CHUNK)) * _CHUNK
_NR_PAD = _RPW * _NW
_NCHUNK = _RPW // _CHUNK
_GROWS = _R * _MPAD
_BIAS_ROW = _GROWS
_LG = _O // 16


def _tc_precompute_body(sig_ref, k_ref, b_ref, out_ref):
    kc = jnp.sum(k_ref[...], axis=0)
    sig = sig_ref[...]
    for r in range(_R):
        out_ref[r * _MPAD:(r + 1) * _MPAD, :] = lax.dot_general(
            sig, kc[r], (((1,), (1,)), ((), ())),
            preferred_element_type=jnp.float32)
    bias = (b_ref.shape[0] * _R) * jnp.sum(b_ref[...], axis=0)
    out_ref[_BIAS_ROW:_BIAS_ROW + 8, :] = jnp.broadcast_to(bias[None, :], (8, _O))


_tc_precompute = pl.pallas_call(
    _tc_precompute_body,
    out_shape=jax.ShapeDtypeStruct((_GROWS + 8, _O), jnp.float32),
)


_sc_mesh = plsc.VectorSubcoreMesh(core_axis_name="c", subcore_axis_name="s")


@functools.partial(
    pl.kernel,
    out_type=jax.ShapeDtypeStruct((_NR_PAD, _O), jnp.float32),
    mesh=_sc_mesh,
    scratch_types=[
        pltpu.VMEM((_CHUNK * _T,), jnp.int32),
        pltpu.VMEM((_CHUNK * _T + 16,), jnp.float32),
        pltpu.VMEM((_CHUNK * _T, _O), jnp.float32),
        pltpu.VMEM((_CHUNK, _O), jnp.float32),
        pltpu.VMEM((1, _O), jnp.float32),
        pltpu.SemaphoreType.DMA,
    ],
    compiler_params=pltpu.CompilerParams(use_tc_tiling_on_sc=False),
)
def _sc_gather_combine(g_hbm, fidx_hbm, w_hbm, out_hbm,
                       idx_v, w_v, rows_v, out_v, bias_v, sem):
    wid = lax.axis_index("s") * 2 + lax.axis_index("c")
    base_row = wid * _RPW
    pltpu.sync_copy(g_hbm.at[pl.ds(_BIAS_ROW, 1)], bias_v)

    def chunk_body(i, carry):
        row0 = base_row + i * _CHUNK
        e0 = row0 * _T
        pltpu.sync_copy(fidx_hbm.at[pl.ds(e0, _CHUNK * _T)], idx_v)
        pltpu.sync_copy(w_hbm.at[pl.ds(e0, _CHUNK * _T)],
                        w_v.at[pl.ds(0, _CHUNK * _T)])
        pltpu.async_copy(g_hbm.at[idx_v], rows_v, sem).wait()
        for c in range(_CHUNK):
            accs = [bias_v[0, pl.ds(l * 16, 16)] for l in range(_LG)]
            wrow = w_v[pl.ds(c * _T, 16)]
            for t in range(_T):
                wv = wrow[t]
                for l in range(_LG):
                    accs[l] = accs[l] + wv * rows_v[c * _T + t, pl.ds(l * 16, 16)]
            for l in range(_LG):
                out_v[c, pl.ds(l * 16, 16)] = jnp.maximum(accs[l], 0.0)
        pltpu.sync_copy(out_v, out_hbm.at[pl.ds(row0, _CHUNK)])
        return carry

    lax.fori_loop(0, _NCHUNK, chunk_body, 0)


def kernel(signal, barycentric, kernels, biases):
    sig = jnp.pad(signal[0], ((0, _MPAD - _M), (0, 0)))
    kern_rs = kernels.reshape(_A * kernels.shape[1], _R, _O, _N)
    g_ext = _tc_precompute(sig, kern_rs, biases)

    bar = barycentric[0]
    idx = bar[..., 0].astype(jnp.int32)
    wts = bar[..., 1]
    idx = jnp.transpose(idx, (1, 0, 2, 3)).reshape(_NROWS, _T)
    wts = jnp.transpose(wts, (1, 0, 2, 3)).reshape(_NROWS, _T)
    roff = jnp.repeat(jnp.arange(_R, dtype=jnp.int32) * _MPAD, 3)
    fidx = idx + roff[None, :]
    fidx = jnp.pad(fidx, ((0, _NR_PAD - _NROWS), (0, 0))).reshape(-1)
    wflat = jnp.pad(wts, ((0, _NR_PAD - _NROWS), (0, 0))).reshape(-1)

    out = _sc_gather_combine(g_ext, fidx, wflat)
    return out[:_NROWS].reshape(_A, _M, _O)[None]


# speedup vs baseline: 10.8123x; 1.7001x over previous
import functools

import jax
import jax.numpy as jnp
from jax import lax
from jax.experimental import pallas as pl
from jax.experimental.pallas import tpu as pltpu
from jax.experimental.pallas import tpu_sc as plsc

_B, _M, _N, _O, _A, _R = 1, 6890, 64, 64, 6, 5
_MPAD = 6912
_T = _R * 3
_NROWS = _A * _M
_NW = 32
_CHUNK = 8
_RPW = ((_NROWS + _NW * _CHUNK - 1) // (_NW * _CHUNK)) * _CHUNK
_NR_PAD = _RPW * _NW
_NCHUNK = _RPW // _CHUNK
_NPAIR = _NCHUNK // 2
_GROWS = _R * _MPAD
_BIAS_ROW = _GROWS
_LG = _O // 16
_EC = _CHUNK * _T
_EW = _NCHUNK * _EC


def _tc_precompute_body(sig_ref, k_ref, b_ref, out_ref):
    kc = jnp.sum(k_ref[...], axis=0)
    sig = sig_ref[...]
    for r in range(_R):
        out_ref[r * _MPAD:(r + 1) * _MPAD, :] = lax.dot_general(
            sig, kc[r], (((1,), (1,)), ((), ())),
            preferred_element_type=jnp.float32)
    bias = (b_ref.shape[0] * _R) * jnp.sum(b_ref[...], axis=0)
    out_ref[_BIAS_ROW:_BIAS_ROW + 8, :] = jnp.broadcast_to(bias[None, :], (8, _O))


_tc_precompute = pl.pallas_call(
    _tc_precompute_body,
    out_shape=jax.ShapeDtypeStruct((_GROWS + 8, _O), jnp.float32),
)


_sc_mesh = plsc.VectorSubcoreMesh(core_axis_name="c", subcore_axis_name="s")


@functools.partial(
    pl.kernel,
    out_type=jax.ShapeDtypeStruct((_NR_PAD, _O), jnp.float32),
    mesh=_sc_mesh,
    scratch_types=[
        pltpu.VMEM((_EW + _EC,), jnp.int32),
        pltpu.VMEM((_EW + 16,), jnp.float32),
        pltpu.VMEM((_EC, _O), jnp.float32),
        pltpu.VMEM((_EC, _O), jnp.float32),
        pltpu.VMEM((_CHUNK, _O), jnp.float32),
        pltpu.VMEM((_CHUNK, _O), jnp.float32),
        pltpu.VMEM((1, _O), jnp.float32),
        pltpu.SemaphoreType.DMA,
        pltpu.SemaphoreType.DMA,
        pltpu.SemaphoreType.DMA,
        pltpu.SemaphoreType.DMA,
    ],
    compiler_params=pltpu.CompilerParams(use_tc_tiling_on_sc=False),
)
def _sc_gather_combine(g_hbm, fidx_hbm, w_hbm, out_hbm,
                       idx_all, w_all, rows_a, rows_b, out_a, out_b, bias_v,
                       sem_ga, sem_gb, sem_oa, sem_ob):
    wid = lax.axis_index("s") * 2 + lax.axis_index("c")
    base_row = wid * _RPW
    e0 = base_row * _T
    pltpu.sync_copy(g_hbm.at[pl.ds(_BIAS_ROW, 1)], bias_v)
    pltpu.sync_copy(fidx_hbm.at[pl.ds(e0, _EW + _EC)],
                    idx_all.at[pl.ds(0, _EW + _EC)])
    pltpu.sync_copy(w_hbm.at[pl.ds(e0, _EW)], w_all.at[pl.ds(0, _EW)])

    def gather(i, rows_v, sem):
        src = g_hbm.at[idx_all.at[pl.ds(i * _EC, _EC)]]
        return pltpu.async_copy(src, rows_v, sem)

    def gather_wait(rows_v, sem):
        pltpu.make_async_copy(g_hbm.at[idx_all.at[pl.ds(0, _EC)]],
                              rows_v, sem).wait()

    def compute_chunk(i, rows_v, out_v, out_sem, pending):
        @pl.when(pending)
        def _():
            pltpu.make_async_copy(out_v, out_hbm.at[pl.ds(0, _CHUNK)],
                                  out_sem).wait()
        for c in range(_CHUNK):
            wrow = w_all[pl.ds(i * _EC + c * _T, 16)]
            accs = [bias_v[0, pl.ds(l * 16, 16)] for l in range(_LG)]
            for t in range(_T):
                wv = wrow[t]
                for l in range(_LG):
                    accs[l] = accs[l] + wv * rows_v[c * _T + t, pl.ds(l * 16, 16)]
            for l in range(_LG):
                out_v[c, pl.ds(l * 16, 16)] = jnp.maximum(accs[l], 0.0)
        pltpu.async_copy(out_v, out_hbm.at[pl.ds(base_row + i * _CHUNK, _CHUNK)],
                         out_sem)

    gather(0, rows_a, sem_ga)

    def pair_body(p, carry):
        i0 = 2 * p
        gather(i0 + 1, rows_b, sem_gb)
        gather_wait(rows_a, sem_ga)
        compute_chunk(i0, rows_a, out_a, sem_oa, p >= 1)
        gather(i0 + 2, rows_a, sem_ga)
        gather_wait(rows_b, sem_gb)
        compute_chunk(i0 + 1, rows_b, out_b, sem_ob, p >= 1)
        return carry

    lax.fori_loop(0, _NPAIR, pair_body, 0)
    gather_wait(rows_a, sem_ga)
    pltpu.make_async_copy(out_a, out_hbm.at[pl.ds(0, _CHUNK)], sem_oa).wait()
    pltpu.make_async_copy(out_b, out_hbm.at[pl.ds(0, _CHUNK)], sem_ob).wait()


def kernel(signal, barycentric, kernels, biases):
    sig = jnp.pad(signal[0], ((0, _MPAD - _M), (0, 0)))
    kern_rs = kernels.reshape(_A * kernels.shape[1], _R, _O, _N)
    g_ext = _tc_precompute(sig, kern_rs, biases)

    bar = barycentric[0]
    idx = bar[..., 0].astype(jnp.int32)
    wts = bar[..., 1]
    idx = jnp.transpose(idx, (1, 0, 2, 3)).reshape(_NROWS, _T)
    wts = jnp.transpose(wts, (1, 0, 2, 3)).reshape(_NROWS, _T)
    roff = jnp.repeat(jnp.arange(_R, dtype=jnp.int32) * _MPAD, 3)
    fidx = idx + roff[None, :]
    fidx = jnp.pad(fidx, ((0, _NR_PAD - _NROWS), (0, 0))).reshape(-1)
    fidx = jnp.pad(fidx, (0, _EC))
    wflat = jnp.pad(wts, ((0, _NR_PAD - _NROWS), (0, 0))).reshape(-1)

    out = _sc_gather_combine(g_ext, fidx, wflat)
    return out[:_NROWS].reshape(_A, _M, _O)[None]
